# baseline (speedup 1.0000x reference)
import functools

import jax
import jax.numpy as jnp
from jax import lax
from jax.experimental import pallas as pl
from jax.experimental.pallas import tpu as pltpu
from jax.experimental.pallas import tpu_sc as plsc

B = 16384
D = 4096
NC = 2
NS = 16
NW = NC * NS
TROWS = B // NW
SUB = 8
NSUB = TROWS // SUB
NBLK = B // SUB
CH = 8
LANES = 16
DCH = D // LANES


def _extract_i32(vec, lane_idx):
    sel = lax.iota(jnp.int32, LANES) == lane_idx
    return jnp.sum(jnp.where(sel, vec, 0))


def _extract_f32(vec, lane_idx):
    sel = lax.iota(jnp.int32, LANES) == lane_idx
    return jnp.sum(jnp.where(sel, vec, jnp.float32(0)))


def _sc_body(rows_hbm, cols_hbm, vals_hbm, k_hbm, bias_hbm, bnd_hbm,
             out_hbm, bias_v, bnd_v, idx_v, row_v, val_v, kbuf, acc, sem):
    w = lax.axis_index("s") * NC + lax.axis_index("c")
    pltpu.sync_copy(bias_hbm, bias_v)
    pltpu.sync_copy(bnd_hbm.at[pl.ds(w * NSUB, 80)], bnd_v)

    def sub_block(b, carry):
        gbase = w * TROWS + b * SUB
        bvec0 = bnd_v[pl.ds((b // LANES) * LANES, LANES)]
        bvec1 = bnd_v[pl.ds(((b + 1) // LANES) * LANES, LANES)]
        s = _extract_i32(bvec0, b % LANES)
        e = _extract_i32(bvec1, (b + 1) % LANES)

        def init_chunk(c, _):
            bv = bias_v[pl.ds(c * LANES, LANES)]
            for r in range(SUB):
                acc[pl.ds(r * D + c * LANES, LANES)] = bv
            return _
        lax.fori_loop(0, DCH, init_chunk, 0)

        k0 = (s // CH) * CH
        nck = (e - k0 + CH - 1) // CH

        def chunk(ci, _):
            base = k0 + ci * CH
            pltpu.sync_copy(cols_hbm.at[pl.ds(base, CH)], idx_v)
            pltpu.sync_copy(rows_hbm.at[pl.ds(base, CH)],
                            row_v.at[pl.ds(0, CH)])
            pltpu.sync_copy(vals_hbm.at[pl.ds(base, CH)],
                            val_v.at[pl.ds(0, CH)])
            pltpu.async_copy(k_hbm.at[idx_v], kbuf, sem).wait()
            rvec = row_v[...]
            vvec = val_v[...]
            for i in range(CH):
                pos = base + i
                row_i = _extract_i32(rvec, i)
                val_i = _extract_f32(vvec, i)
                lrow = row_i - gbase
                valid = jnp.logical_and(pos >= s, pos < e)

                @pl.when(valid)
                def _do():
                    def fma(c, _):
                        off = lrow * D + c * LANES
                        acc[pl.ds(off, LANES)] = (
                            acc[pl.ds(off, LANES)]
                            + val_i * kbuf[i, pl.ds(c * LANES, LANES)])
                        return _
                    lax.fori_loop(0, DCH, fma, 0)
            return _
        lax.fori_loop(0, nck, chunk, 0)

        pltpu.sync_copy(acc, out_hbm.at[pl.ds(gbase * D, SUB * D)])
        return carry

    lax.fori_loop(0, NSUB, sub_block, 0)


def kernel(sp_rows, sp_cols, sp_vals, kernel, bias):
    rows = sp_rows.astype(jnp.int32)
    cols = sp_cols.astype(jnp.int32)
    vals = sp_vals.astype(jnp.float32)

    order = jnp.argsort(rows)
    rows_s = jnp.take(rows, order)
    cols_s = jnp.take(cols, order)
    vals_s = jnp.take(vals, order)

    nnz = rows_s.shape[0]
    pad = (-nnz) % CH + CH
    rows_p = jnp.concatenate([rows_s, jnp.full((pad,), B, jnp.int32)])
    cols_p = jnp.concatenate([cols_s, jnp.zeros((pad,), jnp.int32)])
    vals_p = jnp.concatenate([vals_s, jnp.zeros((pad,), jnp.float32)])

    targets = (jnp.arange(NBLK + 1, dtype=jnp.int32) * SUB)
    bounds = jnp.searchsorted(rows_s, targets, side="left").astype(jnp.int32)
    bounds = jnp.concatenate([bounds, jnp.full((15,), nnz, jnp.int32)])

    mesh = plsc.VectorSubcoreMesh(core_axis_name="c", subcore_axis_name="s")
    run = pl.kernel(
        _sc_body, mesh=mesh,
        out_type=jax.ShapeDtypeStruct((B * D,), jnp.float32),
        compiler_params=pltpu.CompilerParams(needs_layout_passes=False),
        scratch_types=[
            pltpu.VMEM((D,), jnp.float32),
            pltpu.VMEM((80,), jnp.int32),
            pltpu.VMEM((CH,), jnp.int32),
            pltpu.VMEM((LANES,), jnp.int32),
            pltpu.VMEM((LANES,), jnp.float32),
            pltpu.VMEM((CH, D), jnp.float32),
            pltpu.VMEM((SUB * D,), jnp.float32),
            pltpu.SemaphoreType.DMA,
        ],
    )
    out_flat = run(rows_p, cols_p, vals_p, kernel, bias, bounds)
    return out_flat.reshape(B, D)


# speedup vs baseline: 1.1243x; 1.1243x over previous
import jax
import jax.numpy as jnp
from jax import lax
from jax.experimental import pallas as pl
from jax.experimental.pallas import tpu as pltpu
from jax.experimental.pallas import tpu_sc as plsc

B = 16384
D = 4096
NC = 2
NS = 16
NW = NC * NS
TROWS = B // NW
SUB = 8
NSUB = TROWS // SUB
NBLK = B // SUB
CH = 8
MB = 64
LANES = 16
DCH = D // LANES
GRP = 8


def _extract_i32(vec, lane_idx):
    sel = lax.iota(jnp.int32, LANES) == lane_idx
    return jnp.sum(jnp.where(sel, vec, 0))


def _extract_f32(vec, lane_idx):
    sel = lax.iota(jnp.int32, LANES) == lane_idx
    return jnp.sum(jnp.where(sel, vec, jnp.float32(0)))


def _sc_body(rows_hbm, cols_hbm, vals_hbm, k_hbm, bias_hbm, bnd_hbm,
             out_hbm, bias_v, bnd_v, cidx, mrow, mval, kbuf, acc,
             sem_a, sem_b):
    w = lax.axis_index("s") * NC + lax.axis_index("c")
    pltpu.sync_copy(bias_hbm, bias_v)
    pltpu.sync_copy(bnd_hbm.at[pl.ds(w * NSUB, 80)], bnd_v)

    def load_meta(mbase, ms):
        pltpu.sync_copy(cols_hbm.at[pl.ds(mbase, MB)],
                        cidx.at[pl.ds(ms * MB, MB)])
        pltpu.sync_copy(rows_hbm.at[pl.ds(mbase, MB)],
                        mrow.at[pl.ds(ms * MB, MB)])
        pltpu.sync_copy(vals_hbm.at[pl.ds(mbase, MB)],
                        mval.at[pl.ds(ms * MB, MB)])

    def issue_gather(ci, k0):
        ms = (ci // GRP) % 2
        rel = ci % GRP
        idxref = cidx.at[pl.ds(ms * MB + rel * CH, CH)]

        @pl.when(ci % 2 == 0)
        def _even():
            pltpu.async_copy(k_hbm.at[idxref], kbuf.at[0], sem_a)

        @pl.when(ci % 2 == 1)
        def _odd():
            pltpu.async_copy(k_hbm.at[idxref], kbuf.at[1], sem_b)

    def wait_gather(ci):
        @pl.when(ci % 2 == 0)
        def _even():
            pltpu.make_async_copy(k_hbm.at[pl.ds(0, CH)], kbuf.at[0],
                                  sem_a).wait()

        @pl.when(ci % 2 == 1)
        def _odd():
            pltpu.make_async_copy(k_hbm.at[pl.ds(0, CH)], kbuf.at[1],
                                  sem_b).wait()

    def sub_block(b, carry):
        gbase = w * TROWS + b * SUB
        bvec0 = bnd_v[pl.ds((b // LANES) * LANES, LANES)]
        bvec1 = bnd_v[pl.ds(((b + 1) // LANES) * LANES, LANES)]
        s = _extract_i32(bvec0, b % LANES)
        e = _extract_i32(bvec1, (b + 1) % LANES)

        def init_grp(g, _):
            for u in range(GRP):
                c = g * GRP + u
                bv = bias_v[pl.ds(c * LANES, LANES)]
                for r in range(SUB):
                    acc[pl.ds(r * D + c * LANES, LANES)] = bv
            return _
        lax.fori_loop(0, DCH // GRP, init_grp, 0)

        k0 = (s // CH) * CH
        nck = (e - k0 + CH - 1) // CH

        @pl.when(nck > 0)
        def _prologue():
            load_meta(k0, 0)
            issue_gather(0, k0)

        def chunk(ci, _):
            nxt = ci + 1

            @pl.when(nxt < nck)
            def _prefetch():
                @pl.when(nxt % GRP == 0)
                def _meta():
                    load_meta(k0 + nxt * CH, (nxt // GRP) % 2)
                issue_gather(nxt, k0)

            wait_gather(ci)
            ks = ci % 2
            moff0 = ((ci // GRP) % 2) * MB + (ci % GRP) * CH
            for i in range(CH):
                pos = k0 + ci * CH + i
                moff = moff0 + i
                vbase = (moff // LANES) * LANES
                lane = moff % LANES
                rvec = mrow[pl.ds(vbase, LANES)]
                vvec = mval[pl.ds(vbase, LANES)]
                row_i = _extract_i32(rvec, lane)
                val_i = _extract_f32(vvec, lane)
                lrow = row_i - gbase
                valid = jnp.logical_and(pos >= s, pos < e)

                @pl.when(valid)
                def _do():
                    abase = lrow * D
                    def fma_grp(g, _):
                        for u in range(GRP):
                            c = g * GRP + u
                            acc[pl.ds(abase + c * LANES, LANES)] = (
                                acc[pl.ds(abase + c * LANES, LANES)]
                                + val_i * kbuf[ks, i,
                                               pl.ds(c * LANES, LANES)])
                        return _
                    lax.fori_loop(0, DCH // GRP, fma_grp, 0)
            return _
        lax.fori_loop(0, nck, chunk, 0)

        pltpu.sync_copy(acc, out_hbm.at[pl.ds(gbase * D, SUB * D)])
        return carry

    lax.fori_loop(0, NSUB, sub_block, 0)


def kernel(sp_rows, sp_cols, sp_vals, kernel, bias):
    rows = sp_rows.astype(jnp.int32)
    cols = sp_cols.astype(jnp.int32)
    vals = sp_vals.astype(jnp.float32)

    order = jnp.argsort(rows)
    rows_s = jnp.take(rows, order)
    cols_s = jnp.take(cols, order)
    vals_s = jnp.take(vals, order)

    nnz = rows_s.shape[0]
    pad = (-nnz) % CH + 2 * MB
    rows_p = jnp.concatenate([rows_s, jnp.full((pad,), B, jnp.int32)])
    cols_p = jnp.concatenate([cols_s, jnp.zeros((pad,), jnp.int32)])
    vals_p = jnp.concatenate([vals_s, jnp.zeros((pad,), jnp.float32)])

    targets = (jnp.arange(NBLK + 1, dtype=jnp.int32) * SUB)
    bounds = jnp.searchsorted(rows_s, targets, side="left").astype(jnp.int32)
    bounds = jnp.concatenate([bounds, jnp.full((15,), nnz, jnp.int32)])

    mesh = plsc.VectorSubcoreMesh(core_axis_name="c", subcore_axis_name="s")
    run = pl.kernel(
        _sc_body, mesh=mesh,
        out_type=jax.ShapeDtypeStruct((B * D,), jnp.float32),
        compiler_params=pltpu.CompilerParams(needs_layout_passes=False),
        scratch_types=[
            pltpu.VMEM((D,), jnp.float32),
            pltpu.VMEM((80,), jnp.int32),
            pltpu.VMEM((2 * MB,), jnp.int32),
            pltpu.VMEM((2 * MB,), jnp.int32),
            pltpu.VMEM((2 * MB,), jnp.float32),
            pltpu.VMEM((2, CH, D), jnp.float32),
            pltpu.VMEM((SUB * D,), jnp.float32),
            pltpu.SemaphoreType.DMA,
            pltpu.SemaphoreType.DMA,
        ],
    )
    out_flat = run(rows_p, cols_p, vals_p, kernel, bias, bounds)
    return out_flat.reshape(B, D)


# speedup vs baseline: 2.2313x; 1.9846x over previous
import jax
import jax.numpy as jnp
from jax import lax
from jax.experimental import pallas as pl
from jax.experimental.pallas import tpu as pltpu
from jax.experimental.pallas import tpu_sc as plsc

B = 16384
D = 4096
NC = 2
NS = 16
NW = NC * NS
TROWS = B // NW
SUB = 8
NSUB = TROWS // SUB
NBLK = B // SUB
CH = 8
MB = 64
LANES = 16
DCH = D // LANES
GRP = 8


def _extract_i32(vec, lane_idx):
    sel = lax.iota(jnp.int32, LANES) == lane_idx
    return jnp.sum(jnp.where(sel, vec, 0))


def _extract_f32(vec, lane_idx):
    sel = lax.iota(jnp.int32, LANES) == lane_idx
    return jnp.sum(jnp.where(sel, vec, jnp.float32(0)))


def _sc_body(rows_hbm, cols_hbm, vals_hbm, k_hbm, bias_hbm, bnd_hbm,
             out_hbm, bias_v, bnd_v, cidx, mrow, mval, kbuf, acc,
             sem_a, sem_b):
    w = lax.axis_index("s") * NC + lax.axis_index("c")
    pltpu.sync_copy(bias_hbm, bias_v)
    pltpu.sync_copy(bnd_hbm.at[pl.ds(w * NSUB, 80)], bnd_v)

    def load_meta(mbase, ms):
        pltpu.sync_copy(cols_hbm.at[pl.ds(mbase, MB)],
                        cidx.at[pl.ds(ms * MB, MB)])
        pltpu.sync_copy(rows_hbm.at[pl.ds(mbase, MB)],
                        mrow.at[pl.ds(ms * MB, MB)])
        pltpu.sync_copy(vals_hbm.at[pl.ds(mbase, MB)],
                        mval.at[pl.ds(ms * MB, MB)])

    def issue_gather(ci, k0):
        ms = (ci // GRP) % 2
        rel = ci % GRP
        idxref = cidx.at[pl.ds(ms * MB + rel * CH, CH)]

        @pl.when(ci % 2 == 0)
        def _even():
            pltpu.async_copy(k_hbm.at[idxref], kbuf.at[0], sem_a)

        @pl.when(ci % 2 == 1)
        def _odd():
            pltpu.async_copy(k_hbm.at[idxref], kbuf.at[1], sem_b)

    def wait_gather(ci):
        @pl.when(ci % 2 == 0)
        def _even():
            pltpu.make_async_copy(k_hbm.at[pl.ds(0, CH)], kbuf.at[0],
                                  sem_a).wait()

        @pl.when(ci % 2 == 1)
        def _odd():
            pltpu.make_async_copy(k_hbm.at[pl.ds(0, CH)], kbuf.at[1],
                                  sem_b).wait()

    def sub_block(b, carry):
        gbase = w * TROWS + b * SUB
        bvec0 = bnd_v[pl.ds((b // LANES) * LANES, LANES)]
        bvec1 = bnd_v[pl.ds(((b + 1) // LANES) * LANES, LANES)]
        s = _extract_i32(bvec0, b % LANES)
        e = _extract_i32(bvec1, (b + 1) % LANES)

        @plsc.parallel_loop(0, DCH, 1, unroll=GRP)
        def _init(c):
            bv = bias_v[pl.ds(c * LANES, LANES)]
            for r in range(SUB):
                acc[pl.ds(r * D + c * LANES, LANES)] = bv

        k0 = (s // CH) * CH
        nck = (e - k0 + CH - 1) // CH

        @pl.when(nck > 0)
        def _prologue():
            load_meta(k0, 0)
            issue_gather(0, k0)

        def chunk(ci, _):
            nxt = ci + 1

            @pl.when(nxt < nck)
            def _prefetch():
                @pl.when(nxt % GRP == 0)
                def _meta():
                    load_meta(k0 + nxt * CH, (nxt // GRP) % 2)
                issue_gather(nxt, k0)

            wait_gather(ci)
            ks = ci % 2
            moff0 = ((ci // GRP) % 2) * MB + (ci % GRP) * CH
            for i in range(CH):
                pos = k0 + ci * CH + i
                moff = moff0 + i
                vbase = (moff // LANES) * LANES
                lane = moff % LANES
                rvec = mrow[pl.ds(vbase, LANES)]
                vvec = mval[pl.ds(vbase, LANES)]
                row_i = _extract_i32(rvec, lane)
                val_i = _extract_f32(vvec, lane)
                lrow = row_i - gbase
                valid = jnp.logical_and(pos >= s, pos < e)

                @pl.when(valid)
                def _do():
                    abase = lrow * D

                    @plsc.parallel_loop(0, DCH, 1, unroll=GRP)
                    def _fma(c):
                        acc[pl.ds(abase + c * LANES, LANES)] = (
                            acc[pl.ds(abase + c * LANES, LANES)]
                            + val_i * kbuf[ks, i, pl.ds(c * LANES, LANES)])
            return _
        lax.fori_loop(0, nck, chunk, 0)

        pltpu.sync_copy(acc, out_hbm.at[pl.ds(gbase * D, SUB * D)])
        return carry

    lax.fori_loop(0, NSUB, sub_block, 0)


def kernel(sp_rows, sp_cols, sp_vals, kernel, bias):
    rows = sp_rows.astype(jnp.int32)
    cols = sp_cols.astype(jnp.int32)
    vals = sp_vals.astype(jnp.float32)

    order = jnp.argsort(rows)
    rows_s = jnp.take(rows, order)
    cols_s = jnp.take(cols, order)
    vals_s = jnp.take(vals, order)

    nnz = rows_s.shape[0]
    pad = (-nnz) % CH + 2 * MB
    rows_p = jnp.concatenate([rows_s, jnp.full((pad,), B, jnp.int32)])
    cols_p = jnp.concatenate([cols_s, jnp.zeros((pad,), jnp.int32)])
    vals_p = jnp.concatenate([vals_s, jnp.zeros((pad,), jnp.float32)])

    targets = (jnp.arange(NBLK + 1, dtype=jnp.int32) * SUB)
    bounds = jnp.searchsorted(rows_s, targets, side="left").astype(jnp.int32)
    bounds = jnp.concatenate([bounds, jnp.full((15,), nnz, jnp.int32)])

    mesh = plsc.VectorSubcoreMesh(core_axis_name="c", subcore_axis_name="s")
    run = pl.kernel(
        _sc_body, mesh=mesh,
        out_type=jax.ShapeDtypeStruct((B * D,), jnp.float32),
        compiler_params=pltpu.CompilerParams(needs_layout_passes=False),
        scratch_types=[
            pltpu.VMEM((D,), jnp.float32),
            pltpu.VMEM((80,), jnp.int32),
            pltpu.VMEM((2 * MB,), jnp.int32),
            pltpu.VMEM((2 * MB,), jnp.int32),
            pltpu.VMEM((2 * MB,), jnp.float32),
            pltpu.VMEM((2, CH, D), jnp.float32),
            pltpu.VMEM((SUB * D,), jnp.float32),
            pltpu.SemaphoreType.DMA,
            pltpu.SemaphoreType.DMA,
        ],
    )
    out_flat = run(rows_p, cols_p, vals_p, kernel, bias, bounds)
    return out_flat.reshape(B, D)


# speedup vs baseline: 3.5571x; 1.5942x over previous
import jax
import jax.numpy as jnp
from jax import lax
from jax.experimental import pallas as pl
from jax.experimental.pallas import tpu as pltpu
from jax.experimental.pallas import tpu_sc as plsc

B = 16384
D = 4096
NC = 2
NS = 16
NW = NC * NS
TROWS = B // NW
SUB = 8
NSUB = TROWS // SUB
NBLK = B // SUB
CH = 8
MB = 64
LANES = 16
DCH = D // LANES
GRP = 8


def _extract_i32(vec, lane_idx):
    sel = lax.iota(jnp.int32, LANES) == lane_idx
    return jnp.sum(jnp.where(sel, vec, 0))


def _extract_f32(vec, lane_idx):
    sel = lax.iota(jnp.int32, LANES) == lane_idx
    return jnp.sum(jnp.where(sel, vec, jnp.float32(0)))


def _sc_body(rows_hbm, cols_hbm, vals_hbm, k_hbm, bias_hbm, bnd_hbm,
             out_hbm, bias_v, bnd_v, cidx, mrow, mval, kbuf, acc,
             sem_a, sem_b):
    w = lax.axis_index("s") * NC + lax.axis_index("c")
    pltpu.sync_copy(bias_hbm, bias_v)
    pltpu.sync_copy(bnd_hbm.at[pl.ds(w * NSUB, 80)], bnd_v)

    def load_meta(mbase, ms):
        pltpu.sync_copy(cols_hbm.at[pl.ds(mbase, MB)],
                        cidx.at[pl.ds(ms * MB, MB)])
        pltpu.sync_copy(rows_hbm.at[pl.ds(mbase, MB)],
                        mrow.at[pl.ds(ms * MB, MB)])
        pltpu.sync_copy(vals_hbm.at[pl.ds(mbase, MB)],
                        mval.at[pl.ds(ms * MB, MB)])

    def issue_gather(ci, k0):
        ms = (ci // GRP) % 2
        rel = ci % GRP
        idxref = cidx.at[pl.ds(ms * MB + rel * CH, CH)]

        @pl.when(ci % 2 == 0)
        def _even():
            pltpu.async_copy(k_hbm.at[idxref], kbuf.at[0], sem_a)

        @pl.when(ci % 2 == 1)
        def _odd():
            pltpu.async_copy(k_hbm.at[idxref], kbuf.at[1], sem_b)

    def wait_gather(ci):
        @pl.when(ci % 2 == 0)
        def _even():
            pltpu.make_async_copy(k_hbm.at[pl.ds(0, CH)], kbuf.at[0],
                                  sem_a).wait()

        @pl.when(ci % 2 == 1)
        def _odd():
            pltpu.make_async_copy(k_hbm.at[pl.ds(0, CH)], kbuf.at[1],
                                  sem_b).wait()

    def sub_block(b, carry):
        gbase = w * TROWS + b * SUB
        bvec0 = bnd_v[pl.ds((b // LANES) * LANES, LANES)]
        bvec1 = bnd_v[pl.ds(((b + 1) // LANES) * LANES, LANES)]
        s = _extract_i32(bvec0, b % LANES)
        e = _extract_i32(bvec1, (b + 1) % LANES)

        @plsc.parallel_loop(0, DCH, 1, unroll=GRP)
        def _init(c):
            bv = bias_v[pl.ds(c * LANES, LANES)]
            for r in range(SUB):
                acc[r, pl.ds(c * LANES, LANES)] = bv

        k0 = (s // CH) * CH
        nck = (e - k0 + CH - 1) // CH

        @pl.when(nck > 0)
        def _prologue():
            load_meta(k0, 0)
            issue_gather(0, k0)

        def chunk(ci, _):
            nxt = ci + 1

            @pl.when(nxt < nck)
            def _prefetch():
                @pl.when(nxt % GRP == 0)
                def _meta():
                    load_meta(k0 + nxt * CH, (nxt // GRP) % 2)
                issue_gather(nxt, k0)

            wait_gather(ci)
            ks = ci % 2
            moff0 = ((ci // GRP) % 2) * MB + (ci % GRP) * CH
            for i in range(CH):
                pos = k0 + ci * CH + i
                moff = moff0 + i
                vbase = (moff // LANES) * LANES
                lane = moff % LANES
                rvec = mrow[pl.ds(vbase, LANES)]
                vvec = mval[pl.ds(vbase, LANES)]
                row_i = _extract_i32(rvec, lane)
                val_i = _extract_f32(vvec, lane)
                lrow = row_i - gbase
                valid = jnp.logical_and(pos >= s, pos < e)

                @pl.when(valid)
                def _do():
                    @plsc.parallel_loop(0, DCH, 1, unroll=GRP)
                    def _fma(c):
                        acc[lrow, pl.ds(c * LANES, LANES)] = (
                            acc[lrow, pl.ds(c * LANES, LANES)]
                            + val_i * kbuf[ks, i, pl.ds(c * LANES, LANES)])
            return _
        lax.fori_loop(0, nck, chunk, 0)

        pltpu.sync_copy(acc, out_hbm.at[pl.ds(gbase, SUB)])
        return carry

    lax.fori_loop(0, NSUB, sub_block, 0)


def kernel(sp_rows, sp_cols, sp_vals, kernel, bias):
    rows = sp_rows.astype(jnp.int32)
    cols = sp_cols.astype(jnp.int32)
    vals = sp_vals.astype(jnp.float32)

    rows_s, cols_s, vals_s = lax.sort((rows, cols, vals), num_keys=1)

    nnz = rows_s.shape[0]
    pad = (-nnz) % CH + 2 * MB
    rows_p = jnp.concatenate([rows_s, jnp.full((pad,), B, jnp.int32)])
    cols_p = jnp.concatenate([cols_s, jnp.zeros((pad,), jnp.int32)])
    vals_p = jnp.concatenate([vals_s, jnp.zeros((pad,), jnp.float32)])

    targets = (jnp.arange(NBLK + 1, dtype=jnp.int32) * SUB)
    bounds = jnp.searchsorted(rows_s, targets, side="left",
                              method="sort").astype(jnp.int32)
    bounds = jnp.concatenate([bounds, jnp.full((15,), nnz, jnp.int32)])

    mesh = plsc.VectorSubcoreMesh(core_axis_name="c", subcore_axis_name="s")
    run = pl.kernel(
        _sc_body, mesh=mesh,
        out_type=jax.ShapeDtypeStruct((B, D), jnp.float32),
        compiler_params=pltpu.CompilerParams(needs_layout_passes=False),
        scratch_types=[
            pltpu.VMEM((D,), jnp.float32),
            pltpu.VMEM((80,), jnp.int32),
            pltpu.VMEM((2 * MB,), jnp.int32),
            pltpu.VMEM((2 * MB,), jnp.int32),
            pltpu.VMEM((2 * MB,), jnp.float32),
            pltpu.VMEM((2, CH, D), jnp.float32),
            pltpu.VMEM((SUB, D), jnp.float32),
            pltpu.SemaphoreType.DMA,
            pltpu.SemaphoreType.DMA,
        ],
    )
    return run(rows_p, cols_p, vals_p, kernel, bias, bounds)


# speedup vs baseline: 4.3800x; 1.2314x over previous
import jax
import jax.numpy as jnp
from jax import lax
from jax.experimental import pallas as pl
from jax.experimental.pallas import tpu as pltpu
from jax.experimental.pallas import tpu_sc as plsc

B = 16384
D = 4096
NC = 2
NS = 16
NW = NC * NS
TROWS = B // NW
SUB = 8
NSUB = TROWS // SUB
NBLK = B // SUB
CH = 8
MB = 64
LANES = 16
DCH = D // LANES
GRP = 8


def _extract_i32(vec, lane_idx):
    sel = lax.iota(jnp.int32, LANES) == lane_idx
    return jnp.sum(jnp.where(sel, vec, 0))


_GDN = lax.GatherDimensionNumbers(offset_dims=(), collapsed_slice_dims=(0,),
                                  start_index_map=(0,))


def _bcast_lane(vec, lane_idx):
    idx = jnp.full((LANES, 1), 0, jnp.int32) + lane_idx
    return lax.gather(vec, idx, _GDN, (1,),
                      mode=lax.GatherScatterMode.PROMISE_IN_BOUNDS)


def _sc_body(rows_hbm, cols_hbm, vals_hbm, k_hbm, bias_hbm, bnd_hbm,
             out_hbm, bias_v, bnd_v, cidx, mrow, mval, kbuf, acc,
             sem_a, sem_b):
    w = lax.axis_index("s") * NC + lax.axis_index("c")
    pltpu.sync_copy(bias_hbm, bias_v)
    pltpu.sync_copy(bnd_hbm.at[pl.ds(w * NSUB, 80)], bnd_v)

    def load_meta(mbase, ms):
        pltpu.sync_copy(cols_hbm.at[pl.ds(mbase, MB)],
                        cidx.at[pl.ds(ms * MB, MB)])
        pltpu.sync_copy(rows_hbm.at[pl.ds(mbase, MB)],
                        mrow.at[pl.ds(ms * MB, MB)])
        pltpu.sync_copy(vals_hbm.at[pl.ds(mbase, MB)],
                        mval.at[pl.ds(ms * MB, MB)])

    def issue_gather(ci, k0):
        ms = (ci // GRP) % 2
        rel = ci % GRP
        idxref = cidx.at[pl.ds(ms * MB + rel * CH, CH)]

        @pl.when(ci % 2 == 0)
        def _even():
            pltpu.async_copy(k_hbm.at[idxref], kbuf.at[0], sem_a)

        @pl.when(ci % 2 == 1)
        def _odd():
            pltpu.async_copy(k_hbm.at[idxref], kbuf.at[1], sem_b)

    def wait_gather(ci):
        @pl.when(ci % 2 == 0)
        def _even():
            pltpu.make_async_copy(k_hbm.at[pl.ds(0, CH)], kbuf.at[0],
                                  sem_a).wait()

        @pl.when(ci % 2 == 1)
        def _odd():
            pltpu.make_async_copy(k_hbm.at[pl.ds(0, CH)], kbuf.at[1],
                                  sem_b).wait()

    def sub_block(b, carry):
        gbase = w * TROWS + b * SUB
        bvec0 = bnd_v[pl.ds((b // LANES) * LANES, LANES)]
        bvec1 = bnd_v[pl.ds(((b + 1) // LANES) * LANES, LANES)]
        s = _extract_i32(bvec0, b % LANES)
        e = _extract_i32(bvec1, (b + 1) % LANES)

        @plsc.parallel_loop(0, DCH, 1, unroll=GRP)
        def _init(c):
            bv = bias_v[pl.ds(c * LANES, LANES)]
            for r in range(SUB):
                acc[r, pl.ds(c * LANES, LANES)] = bv

        k0 = (s // CH) * CH
        nck = (e - k0 + CH - 1) // CH

        @pl.when(nck > 0)
        def _prologue():
            load_meta(k0, 0)
            issue_gather(0, k0)

        def chunk(ci, _):
            nxt = ci + 1

            @pl.when(nxt < nck)
            def _prefetch():
                @pl.when(nxt % GRP == 0)
                def _meta():
                    load_meta(k0 + nxt * CH, (nxt // GRP) % 2)
                issue_gather(nxt, k0)

            wait_gather(ci)
            ks = ci % 2
            moff0 = ((ci // GRP) % 2) * MB + (ci % GRP) * CH
            vbase = (moff0 // LANES) * LANES
            lane0 = moff0 % LANES
            rvec = mrow[pl.ds(vbase, LANES)]
            vvec = mval[pl.ds(vbase, LANES)]
            ioti = lax.iota(jnp.int32, LANES)
            for i in range(CH):
                pos = k0 + ci * CH + i
                vval = _bcast_lane(vvec, lane0 + i)
                vrow = _bcast_lane(rvec, lane0 + i)
                rowidx = vrow - gbase
                valid = jnp.logical_and(pos >= s, pos < e)

                @pl.when(valid)
                def _do():
                    @plsc.parallel_loop(0, DCH, 1, unroll=GRP)
                    def _fma(c):
                        colidx = ioti + c * LANES
                        kv = kbuf[ks, i, pl.ds(c * LANES, LANES)]
                        ov = plsc.load_gather(acc, [rowidx, colidx])
                        plsc.store_scatter(acc, [rowidx, colidx],
                                           ov + vval * kv)
            return _
        lax.fori_loop(0, nck, chunk, 0)

        pltpu.sync_copy(acc, out_hbm.at[pl.ds(gbase, SUB)])
        return carry

    lax.fori_loop(0, NSUB, sub_block, 0)


def kernel(sp_rows, sp_cols, sp_vals, kernel, bias):
    rows = sp_rows.astype(jnp.int32)
    cols = sp_cols.astype(jnp.int32)
    vals = sp_vals.astype(jnp.float32)

    key = rows * D + cols
    key_s, vals_s = lax.sort((key, vals), num_keys=1)
    rows_s = key_s // D
    cols_s = key_s - rows_s * D

    nnz = rows_s.shape[0]
    pad = (-nnz) % CH + 2 * MB
    rows_p = jnp.concatenate([rows_s, jnp.full((pad,), B, jnp.int32)])
    cols_p = jnp.concatenate([cols_s, jnp.zeros((pad,), jnp.int32)])
    vals_p = jnp.concatenate([vals_s, jnp.zeros((pad,), jnp.float32)])

    targets = (jnp.arange(NBLK + 1, dtype=jnp.int32) * SUB)
    bounds = jnp.searchsorted(rows_s, targets, side="left").astype(jnp.int32)
    bounds = jnp.concatenate([bounds, jnp.full((15,), nnz, jnp.int32)])

    mesh = plsc.VectorSubcoreMesh(core_axis_name="c", subcore_axis_name="s")
    run = pl.kernel(
        _sc_body, mesh=mesh,
        out_type=jax.ShapeDtypeStruct((B, D), jnp.float32),
        compiler_params=pltpu.CompilerParams(needs_layout_passes=False),
        scratch_types=[
            pltpu.VMEM((D,), jnp.float32),
            pltpu.VMEM((80,), jnp.int32),
            pltpu.VMEM((2 * MB,), jnp.int32),
            pltpu.VMEM((2 * MB,), jnp.int32),
            pltpu.VMEM((2 * MB,), jnp.float32),
            pltpu.VMEM((2, CH, D), jnp.float32),
            pltpu.VMEM((SUB, D), jnp.float32),
            pltpu.SemaphoreType.DMA,
            pltpu.SemaphoreType.DMA,
        ],
    )
    return run(rows_p, cols_p, vals_p, kernel, bias, bounds)


# speedup vs baseline: 5.0720x; 1.1580x over previous
import jax
import jax.numpy as jnp
from jax import lax
from jax.experimental import pallas as pl
from jax.experimental.pallas import tpu as pltpu
from jax.experimental.pallas import tpu_sc as plsc

B = 16384
D = 4096
NC = 2
NS = 16
NW = NC * NS
TROWS = B // NW
SUB = 8
NSUB = TROWS // SUB
CH = 8
MB = 128
CPG = MB // CH
LANES = 16
DCH = D // LANES
GRP = 8


def _extract_i32(vec, lane_idx):
    sel = lax.iota(jnp.int32, LANES) == lane_idx
    return jnp.sum(jnp.where(sel, vec, 0))


_GDN = lax.GatherDimensionNumbers(offset_dims=(), collapsed_slice_dims=(0,),
                                  start_index_map=(0,))


def _bcast_lane(vec, lane_idx):
    idx = jnp.full((LANES, 1), 0, jnp.int32) + lane_idx
    return lax.gather(vec, idx, _GDN, (1,),
                      mode=lax.GatherScatterMode.PROMISE_IN_BOUNDS)


def _sc_body(key_hbm, vals_hbm, k_hbm, bias_hbm, tb_hbm, out_hbm,
             bias_v, tb_v, bnd_v, hist, mkey, mval, cidx, kbuf, acc,
             sem_a, sem_b):
    w = lax.axis_index("s") * NC + lax.axis_index("c")
    pltpu.sync_copy(bias_hbm, bias_v)
    pltpu.sync_copy(tb_hbm, tb_v)
    ioti = lax.iota(jnp.int32, LANES)

    s_w = _extract_i32(tb_v[pl.ds((w // LANES) * LANES, LANES)], w % LANES)
    e_w = _extract_i32(tb_v[pl.ds(((w + 1) // LANES) * LANES, LANES)],
                       (w + 1) % LANES)

    zero16 = jnp.zeros((LANES,), jnp.int32)
    for j in range(4):
        hist[pl.ds(j * LANES, LANES)] = zero16

    k0w = (s_w // CH) * CH
    npre = (e_w - k0w + MB - 1) // MB

    def pre(g, _):
        base = k0w + g * MB
        pltpu.sync_copy(key_hbm.at[pl.ds(base, MB)], mkey.at[pl.ds(0, MB)])
        for j in range(MB // LANES):
            kv = mkey[pl.ds(j * LANES, LANES)]
            pos = base + j * LANES + ioti
            m = jnp.logical_and(pos >= s_w, pos < e_w)
            d = lax.shift_right_logical(kv, 15) - w * NSUB
            d = jnp.clip(d, 0, NSUB - 1)
            one = jnp.where(m, 1, 0)
            plsc.addupdate_scatter(hist, [d], one)
        return _
    lax.fori_loop(0, npre, pre, 0)

    carry = zero16
    for j in range(4):
        h = hist[pl.ds(j * LANES, LANES)]
        inc = plsc.cumsum(h) + carry
        bnd_v[pl.ds(j * LANES, LANES)] = inc - h + s_w
        carry = _bcast_lane(inc, LANES - 1)
    bnd_v[pl.ds(NSUB, LANES)] = zero16 + e_w

    def load_meta(mbase, ms):
        pltpu.sync_copy(key_hbm.at[pl.ds(mbase, MB)],
                        mkey.at[pl.ds(ms * MB, MB)])
        pltpu.sync_copy(vals_hbm.at[pl.ds(mbase, MB)],
                        mval.at[pl.ds(ms * MB, MB)])
        for j in range(MB // LANES):
            kv = mkey[pl.ds(ms * MB + j * LANES, LANES)]
            cidx[pl.ds(ms * MB + j * LANES, LANES)] = (
                jnp.bitwise_and(kv, D - 1))

    def issue_gather(ci, k0):
        ms = (ci // CPG) % 2
        rel = ci % CPG
        idxref = cidx.at[pl.ds(ms * MB + rel * CH, CH)]

        @pl.when(ci % 2 == 0)
        def _even():
            pltpu.async_copy(k_hbm.at[idxref], kbuf.at[0], sem_a)

        @pl.when(ci % 2 == 1)
        def _odd():
            pltpu.async_copy(k_hbm.at[idxref], kbuf.at[1], sem_b)

    def wait_gather(ci):
        @pl.when(ci % 2 == 0)
        def _even():
            pltpu.make_async_copy(k_hbm.at[pl.ds(0, CH)], kbuf.at[0],
                                  sem_a).wait()

        @pl.when(ci % 2 == 1)
        def _odd():
            pltpu.make_async_copy(k_hbm.at[pl.ds(0, CH)], kbuf.at[1],
                                  sem_b).wait()

    def sub_block(b, carry2):
        gbase = w * TROWS + b * SUB
        bvec0 = bnd_v[pl.ds((b // LANES) * LANES, LANES)]
        bvec1 = bnd_v[pl.ds(((b + 1) // LANES) * LANES, LANES)]
        s = _extract_i32(bvec0, b % LANES)
        e = _extract_i32(bvec1, (b + 1) % LANES)

        @plsc.parallel_loop(0, DCH, 1, unroll=GRP)
        def _init(c):
            bv = bias_v[pl.ds(c * LANES, LANES)]
            for r in range(SUB):
                acc[r, pl.ds(c * LANES, LANES)] = bv

        k0 = (s // CH) * CH
        nck = (e - k0 + CH - 1) // CH

        @pl.when(nck > 0)
        def _prologue():
            load_meta(k0, 0)
            issue_gather(0, k0)

        def chunk(ci, _):
            nxt = ci + 1

            @pl.when(nxt < nck)
            def _prefetch():
                @pl.when(nxt % CPG == 0)
                def _meta():
                    load_meta(k0 + nxt * CH, (nxt // CPG) % 2)
                issue_gather(nxt, k0)

            wait_gather(ci)
            ks = ci % 2
            moff0 = ((ci // CPG) % 2) * MB + (ci % CPG) * CH
            vbase = (moff0 // LANES) * LANES
            lane0 = moff0 % LANES
            kvec = mkey[pl.ds(vbase, LANES)]
            vvec = mval[pl.ds(vbase, LANES)]
            for i in range(CH):
                pos = k0 + ci * CH + i
                vval = _bcast_lane(vvec, lane0 + i)
                vkey = _bcast_lane(kvec, lane0 + i)
                rowidx = lax.shift_right_logical(vkey, 12) - gbase
                valid = jnp.logical_and(pos >= s, pos < e)

                @pl.when(valid)
                def _do():
                    @plsc.parallel_loop(0, DCH, 1, unroll=GRP)
                    def _fma(c):
                        colidx = ioti + c * LANES
                        kv = kbuf[ks, i, pl.ds(c * LANES, LANES)]
                        ov = plsc.load_gather(acc, [rowidx, colidx])
                        plsc.store_scatter(acc, [rowidx, colidx],
                                           ov + vval * kv)
            return _
        lax.fori_loop(0, nck, chunk, 0)

        pltpu.sync_copy(acc, out_hbm.at[pl.ds(gbase, SUB)])
        return carry2

    lax.fori_loop(0, NSUB, sub_block, 0)


def kernel(sp_rows, sp_cols, sp_vals, kernel, bias):
    rows = sp_rows.astype(jnp.int32)
    cols = sp_cols.astype(jnp.int32)
    vals = sp_vals.astype(jnp.float32)

    key = rows * D + cols
    key_s, vals_s = lax.sort((key, vals), num_keys=1)

    nnz = key_s.shape[0]
    pad = (-nnz) % CH + 2 * MB
    key_p = jnp.concatenate([key_s, jnp.full((pad,), B * D, jnp.int32)])
    vals_p = jnp.concatenate([vals_s, jnp.zeros((pad,), jnp.float32)])

    targets = jnp.arange(NW + 1, dtype=jnp.int32) * (TROWS * D)
    tb = jnp.searchsorted(key_s, targets, side="left").astype(jnp.int32)
    tb = jnp.concatenate([tb, jnp.full((15,), nnz, jnp.int32)])

    mesh = plsc.VectorSubcoreMesh(core_axis_name="c", subcore_axis_name="s")
    run = pl.kernel(
        _sc_body, mesh=mesh,
        out_type=jax.ShapeDtypeStruct((B, D), jnp.float32),
        compiler_params=pltpu.CompilerParams(needs_layout_passes=False),
        scratch_types=[
            pltpu.VMEM((D,), jnp.float32),
            pltpu.VMEM((48,), jnp.int32),
            pltpu.VMEM((80,), jnp.int32),
            pltpu.VMEM((64,), jnp.int32),
            pltpu.VMEM((2 * MB,), jnp.int32),
            pltpu.VMEM((2 * MB,), jnp.float32),
            pltpu.VMEM((2 * MB,), jnp.int32),
            pltpu.VMEM((2, CH, D), jnp.float32),
            pltpu.VMEM((SUB, D), jnp.float32),
            pltpu.SemaphoreType.DMA,
            pltpu.SemaphoreType.DMA,
        ],
    )
    return run(key_p, vals_p, kernel, bias, tb)
